# baseline (speedup 1.0000x reference)
import jax
import jax.numpy as jnp
from jax import lax
from jax.experimental import pallas as pl
from jax.experimental.pallas import tpu as pltpu
from jax.experimental.pallas import tpu_sc as plsc

NC = 2
NS = 16
NW = NC * NS
LANES = 16

VOCAB = 1000000
EMBED = 64
ROWS = 4096
COLS = 200
B = ROWS * COLS
B_PER_W = B // NW
CHUNK = 128
N_CHUNKS = B_PER_W // CHUNK
SCALE = float(EMBED) ** 0.5


def _body(x_hbm, table_hbm, out_hbm, idx_v, buf_v, gsem, osem):
    wid = lax.axis_index("s") * NC + lax.axis_index("c")
    pltpu.sync_copy(x_hbm.at[wid], idx_v)

    def scale_chunk(b):
        def row_body(r, _):
            for c in range(EMBED // LANES):
                sl = pl.ds(c * LANES, LANES)
                buf_v[b, r, sl] = buf_v[b, r, sl] * SCALE
            return _
        lax.fori_loop(0, CHUNK, row_body, 0, unroll=2)

    def chunk_pair(g, _):
        for b in range(2):
            i = 2 * g + b
            pltpu.async_copy(table_hbm.at[idx_v.at[i]], buf_v.at[b], gsem).wait()
            scale_chunk(b)
            pltpu.sync_copy(buf_v.at[b], out_hbm.at[wid, i])
        return _

    lax.fori_loop(0, N_CHUNKS // 2, chunk_pair, 0)


def kernel(x, table):
    xs = x.reshape(NW, N_CHUNKS, CHUNK).astype(jnp.int32)
    out = pl.kernel(
        _body,
        out_type=jax.ShapeDtypeStruct((NW, N_CHUNKS, CHUNK, EMBED), jnp.float32),
        mesh=plsc.VectorSubcoreMesh(core_axis_name="c", subcore_axis_name="s"),
        scratch_types=[
            pltpu.VMEM((N_CHUNKS, CHUNK), jnp.int32),
            pltpu.VMEM((2, CHUNK, EMBED), jnp.float32),
            pltpu.SemaphoreType.DMA,
            pltpu.SemaphoreType.DMA,
        ],
        compiler_params=pltpu.CompilerParams(use_tc_tiling_on_sc=False),
    )(xs, table)
    return out.reshape(ROWS, COLS, EMBED)


# speedup vs baseline: 1.1622x; 1.1622x over previous
import jax
import jax.numpy as jnp
from jax import lax
from jax.experimental import pallas as pl
from jax.experimental.pallas import tpu as pltpu
from jax.experimental.pallas import tpu_sc as plsc

NC = 2
NS = 16
NW = NC * NS
LANES = 16

VOCAB = 1000000
EMBED = 64
ROWS = 4096
COLS = 200
B = ROWS * COLS
B_PER_W = B // NW
CHUNK = 128
N_CHUNKS = B_PER_W // CHUNK
SUB = 4
BIG = SUB * CHUNK
NG = B_PER_W // BIG
NBUF = 3
SCALE = float(EMBED) ** 0.5


def _body(x_hbm, table_hbm, out_hbm, idx_v, buf_v, gsem, osem):
    wid = lax.axis_index("s") * NC + lax.axis_index("c")
    pltpu.sync_copy(x_hbm.at[wid], idx_v)

    def fire_gathers(g, b):
        for j in range(SUB):
            pltpu.make_async_copy(
                table_hbm.at[idx_v.at[SUB * g + j]],
                buf_v.at[b, pl.ds(j * CHUNK, CHUNK)],
                gsem,
            ).start()

    def drain_gathers(b):
        for j in range(SUB):
            pltpu.make_async_copy(
                table_hbm.at[idx_v.at[j]],
                buf_v.at[b, pl.ds(j * CHUNK, CHUNK)],
                gsem,
            ).wait()

    def scale(b):
        def row_body(r, carry):
            for c in range(EMBED // LANES):
                sl = pl.ds(c * LANES, LANES)
                buf_v[b, r, sl] = buf_v[b, r, sl] * SCALE
            return carry
        lax.fori_loop(0, BIG, row_body, 0, unroll=4)

    def wait_one_scatter():
        pltpu.make_async_copy(buf_v.at[0], out_hbm.at[wid, 0], osem).wait()

    for gg in range(NBUF):
        fire_gathers(gg, gg)

    def stage(g, carry):
        b = lax.rem(g, NBUF)
        drain_gathers(b)
        scale(b)
        pltpu.make_async_copy(buf_v.at[b], out_hbm.at[wid, g], osem).start()

        @pl.when(g + NBUF < NG)
        def _():
            wait_one_scatter()
            fire_gathers(g + NBUF, b)

        return carry

    lax.fori_loop(0, NG, stage, 0)
    for _ in range(NBUF):
        wait_one_scatter()


def kernel(x, table):
    xs = x.reshape(NW, N_CHUNKS, CHUNK).astype(jnp.int32)
    out = pl.kernel(
        _body,
        out_type=jax.ShapeDtypeStruct((NW, NG, BIG, EMBED), jnp.float32),
        mesh=plsc.VectorSubcoreMesh(core_axis_name="c", subcore_axis_name="s"),
        scratch_types=[
            pltpu.VMEM((N_CHUNKS, CHUNK), jnp.int32),
            pltpu.VMEM((NBUF, BIG, EMBED), jnp.float32),
            pltpu.SemaphoreType.DMA,
            pltpu.SemaphoreType.DMA,
        ],
        compiler_params=pltpu.CompilerParams(use_tc_tiling_on_sc=False),
    )(xs, table)
    return out.reshape(ROWS, COLS, EMBED)
